# baseline (speedup 1.0000x reference)
import jax
import jax.numpy as jnp
from jax.experimental import pallas as pl
from jax.experimental.pallas import tpu as pltpu

_NB = 10
_NCACHE = 4


def _ln(x, g, b):
    mu = jnp.mean(x, axis=-1, keepdims=True)
    var = jnp.mean((x - mu) ** 2, axis=-1, keepdims=True)
    return g * (x - mu) * jax.lax.rsqrt(var + 1e-5) + b


def _mega_body(h_ref, x0_ref, x1_ref, whe_ref, bhe_ref, wv_ref, bv_ref,
               tgl_ref, tgr_ref, n1g_ref, n1b_ref, n2g_ref, n2b_ref,
               w1_ref, b1_ref, w2_ref, b2_ref, wret_ref, bret_ref,
               xout_ref, x1out_ref,
               cache_ref, acc_ref, de_ref, x1new_ref, x1v_ref,
               re_ref, ret_ref):
    i = pl.program_id(0)

    @pl.when(i < _NB)
    def _phase1():
        h = h_ref[...]
        dv = jnp.sum(h, axis=1, keepdims=True)
        rv = jax.lax.rsqrt(jnp.maximum(dv, 1.0))
        hb = h.astype(jnp.bfloat16)

        @pl.when((jax.lax.rem(i, 2) == 1) & (i < 2 * _NCACHE))
        def _():
            cache_ref[(i - 1) // 2] = (h * rv).astype(jnp.bfloat16)

        x0s = (x0_ref[...] * rv).astype(jnp.bfloat16)
        contrib = jax.lax.dot_general(
            x0s, hb, (((0,), (0,)), ((), ())),
            preferred_element_type=jnp.float32)
        de_c = jnp.sum(h, axis=0, keepdims=True)

        @pl.when(i == 0)
        def _():
            acc_ref[...] = contrib
            de_ref[...] = de_c

        @pl.when(i != 0)
        def _():
            acc_ref[...] += contrib
            de_ref[...] += de_c

        @pl.when(i == _NB - 1)
        def _k1_epilogue():
            re = jax.lax.rsqrt(jnp.maximum(de_ref[...], 1.0))
            re_ref[...] = re
            accs = acc_ref[...] * re
            msg = jax.lax.dot_general(
                accs, whe_ref[...], (((0,), (0,)), ((), ())),
                preferred_element_type=jnp.float32)
            x1new = x1_ref[...] + msg + bhe_ref[...]
            x1new_ref[...] = x1new
            re_col = jnp.transpose(re)
            x1v_ref[...] = jnp.dot(x1new * re_col, wv_ref[...],
                                   preferred_element_type=jnp.float32
                                   ).astype(jnp.bfloat16)

    def _phase2_tile(hbs, j):
        msgv = jax.lax.dot_general(
            hbs, x1v_ref[...], (((1,), (0,)), ((), ())),
            preferred_element_type=jnp.float32)
        t = x0_ref[...] + tgl_ref[...] * (msgv + bv_ref[...])
        x0l = _ln(t, n1g_ref[...], n1b_ref[...])
        x0g = _ln(x0l, n2g_ref[...], n2b_ref[...])
        pre = jax.lax.dot_general(
            x0g.astype(jnp.bfloat16), w1_ref[...], (((1,), (0,)), ((), ())),
            preferred_element_type=jnp.float32) + b1_ref[...]
        hmid = pre * 0.5 * (1.0 + jax.lax.erf(pre * 0.7071067811865476))
        xout_ref[...] = x0g + jax.lax.dot_general(
            hmid.astype(jnp.bfloat16), w2_ref[...], (((1,), (0,)), ((), ())),
            preferred_element_type=jnp.float32) + b2_ref[...]
        x0ls = x0l.astype(jnp.bfloat16)
        contrib = jax.lax.dot_general(
            x0ls, hbs, (((0,), (0,)), ((), ())),
            preferred_element_type=jnp.float32)

        @pl.when(j == 0)
        def _():
            ret_ref[...] = contrib

        @pl.when(j != 0)
        def _():
            ret_ref[...] += contrib

        @pl.when(j == _NB - 1)
        def _k2_epilogue():
            rets = ret_ref[...] * re_ref[...]
            msg = jax.lax.dot_general(
                rets, wret_ref[...], (((0,), (0,)), ((), ())),
                preferred_element_type=jnp.float32)
            x1out_ref[...] = x1new_ref[...] + tgr_ref[...] * (
                msg + bret_ref[...])

    j = i - _NB
    is_cached = (jax.lax.rem(j, 2) == 1) & (j < 2 * _NCACHE)

    @pl.when((i >= _NB) & is_cached)
    def _phase2_cached():
        _phase2_tile(cache_ref[(j - 1) // 2], j)

    @pl.when((i >= _NB) & jnp.logical_not(is_cached))
    def _phase2_streamed():
        h = h_ref[...]
        dv = jnp.sum(h, axis=1, keepdims=True)
        rv = jax.lax.rsqrt(jnp.maximum(dv, 1.0))
        _phase2_tile((h * rv).astype(jnp.bfloat16), j)


def kernel(x_0, x_1, incidence_1, params):
    N, D = x_0.shape
    M = x_1.shape[0]
    p = params
    f32 = jnp.float32
    bf16 = jnp.bfloat16
    BN = N // _NB
    nb, nc = _NB, _NCACHE

    tgl = jnp.tanh(p["gate_local"]).reshape(1, 1)
    tgr = jnp.tanh(p["gate_return"]).reshape(1, 1)

    def h_idx(i):
        j = i - nb
        j_idx = jnp.where((jax.lax.rem(j, 2) == 1) & (j < 2 * nc),
                          j - 1, j)
        return (jnp.where(i < nb, i, j_idx), 0)

    const = lambda shape: pl.BlockSpec(shape, lambda i: (0,) * len(shape))

    x_out, x1out = pl.pallas_call(
        _mega_body,
        grid=(2 * _NB,),
        in_specs=[
            pl.BlockSpec((BN, M), h_idx),
            pl.BlockSpec((BN, D), lambda i: (jax.lax.rem(i, nb), 0)),
            const((M, D)),
            const((D, D)),
            const((1, D)),
            const((D, D)),
            const((1, D)),
            const((1, 1)),
            const((1, 1)),
            const((1, D)),
            const((1, D)),
            const((1, D)),
            const((1, D)),
            const((D, 2 * D)),
            const((1, 2 * D)),
            const((2 * D, D)),
            const((1, D)),
            const((D, D)),
            const((1, D)),
        ],
        out_specs=[
            pl.BlockSpec(
                (BN, D),
                lambda i: (jnp.where(i < nb, 0, i - nb), 0)),
            const((M, D)),
        ],
        out_shape=[
            jax.ShapeDtypeStruct((N, D), f32),
            jax.ShapeDtypeStruct((M, D), f32),
        ],
        scratch_shapes=[
            pltpu.VMEM((_NCACHE, BN, M), bf16),
            pltpu.VMEM((D, M), f32),
            pltpu.VMEM((1, M), f32),
            pltpu.VMEM((M, D), f32),
            pltpu.VMEM((M, D), bf16),
            pltpu.VMEM((1, M), f32),
            pltpu.VMEM((D, M), f32),
        ],
        compiler_params=pltpu.CompilerParams(
            dimension_semantics=("arbitrary",),
            vmem_limit_bytes=67108864,
        ),
    )(incidence_1, x_0, x_1,
      p["W_he"], p["b_he"].reshape(1, D), p["W_v"], p["b_v"].reshape(1, D),
      tgl, tgr,
      p["n1_g"].reshape(1, D), p["n1_b"].reshape(1, D),
      p["n2_g"].reshape(1, D), p["n2_b"].reshape(1, D),
      p["W1"].astype(bf16), p["b1"].reshape(1, 2 * D),
      p["W2"].astype(bf16), p["b2"].reshape(1, D),
      p["W_ret"], p["b_ret"].reshape(1, D))

    return x_out, x1out


# speedup vs baseline: 1.0073x; 1.0073x over previous
import jax
import jax.numpy as jnp
from jax.experimental import pallas as pl
from jax.experimental.pallas import tpu as pltpu

_NB = 10
_NCACHE = 4


def _ln(x, g, b):
    mu = jnp.mean(x, axis=-1, keepdims=True)
    var = jnp.mean((x - mu) ** 2, axis=-1, keepdims=True)
    return g * (x - mu) * jax.lax.rsqrt(var + 1e-5) + b


def _mega_body(h_ref, x0_ref, x1_ref, whe_ref, bhe_ref, wv_ref, bv_ref,
               tgl_ref, tgr_ref, n1g_ref, n1b_ref, n2g_ref, n2b_ref,
               w1_ref, b1_ref, w2_ref, b2_ref, wret_ref, bret_ref,
               xout_ref, x1out_ref,
               cache_ref, acc_ref, de_ref, x1new_ref, x1v_ref,
               re_ref, ret_ref):
    i = pl.program_id(0)

    @pl.when(i < _NB)
    def _phase1():
        h = h_ref[...]
        dv = jnp.sum(h, axis=1, keepdims=True)
        rv = jax.lax.rsqrt(jnp.maximum(dv, 1.0))
        hb = h.astype(jnp.bfloat16)

        @pl.when(i < _NCACHE)
        def _():
            cache_ref[i] = (h * rv).astype(jnp.bfloat16)

        x0s = (x0_ref[...] * rv).astype(jnp.bfloat16)
        contrib = jax.lax.dot_general(
            x0s, hb, (((0,), (0,)), ((), ())),
            preferred_element_type=jnp.float32)
        de_c = jnp.sum(h, axis=0, keepdims=True)

        @pl.when(i == 0)
        def _():
            acc_ref[...] = contrib
            de_ref[...] = de_c

        @pl.when(i != 0)
        def _():
            acc_ref[...] += contrib
            de_ref[...] += de_c

        @pl.when(i == _NB - 1)
        def _k1_epilogue():
            re = jax.lax.rsqrt(jnp.maximum(de_ref[...], 1.0))
            re_ref[...] = re
            accs = acc_ref[...] * re
            msg = jax.lax.dot_general(
                accs, whe_ref[...], (((0,), (0,)), ((), ())),
                preferred_element_type=jnp.float32)
            x1new = x1_ref[...] + msg + bhe_ref[...]
            x1new_ref[...] = x1new
            re_col = jnp.transpose(re)
            x1v_ref[...] = jnp.dot(x1new * re_col, wv_ref[...],
                                   preferred_element_type=jnp.float32
                                   ).astype(jnp.bfloat16)

    def _phase2_tile(hbs, j):
        msgv = jax.lax.dot_general(
            hbs, x1v_ref[...], (((1,), (0,)), ((), ())),
            preferred_element_type=jnp.float32)
        t = x0_ref[...] + tgl_ref[...] * (msgv + bv_ref[...])
        x0l = _ln(t, n1g_ref[...], n1b_ref[...])
        x0g = _ln(x0l, n2g_ref[...], n2b_ref[...])
        pre = jax.lax.dot_general(
            x0g.astype(jnp.bfloat16), w1_ref[...], (((1,), (0,)), ((), ())),
            preferred_element_type=jnp.float32) + b1_ref[...]
        hmid = pre * 0.5 * (1.0 + jax.lax.erf(pre * 0.7071067811865476))
        xout_ref[...] = x0g + jax.lax.dot_general(
            hmid.astype(jnp.bfloat16), w2_ref[...], (((1,), (0,)), ((), ())),
            preferred_element_type=jnp.float32) + b2_ref[...]
        x0ls = x0l.astype(jnp.bfloat16)
        contrib = jax.lax.dot_general(
            x0ls, hbs, (((0,), (0,)), ((), ())),
            preferred_element_type=jnp.float32)

        @pl.when(j == 0)
        def _():
            ret_ref[...] = contrib

        @pl.when(j != 0)
        def _():
            ret_ref[...] += contrib

        @pl.when(j == _NB - 1)
        def _k2_epilogue():
            rets = ret_ref[...] * re_ref[...]
            msg = jax.lax.dot_general(
                rets, wret_ref[...], (((0,), (0,)), ((), ())),
                preferred_element_type=jnp.float32)
            x1out_ref[...] = x1new_ref[...] + tgr_ref[...] * (
                msg + bret_ref[...])

    @pl.when((i >= _NB) & (i < _NB + _NCACHE))
    def _phase2_cached():
        _phase2_tile(cache_ref[i - _NB], i - _NB)

    @pl.when(i >= _NB + _NCACHE)
    def _phase2_streamed():
        h = h_ref[...]
        dv = jnp.sum(h, axis=1, keepdims=True)
        rv = jax.lax.rsqrt(jnp.maximum(dv, 1.0))
        _phase2_tile((h * rv).astype(jnp.bfloat16), i - _NB)


def kernel(x_0, x_1, incidence_1, params):
    N, D = x_0.shape
    M = x_1.shape[0]
    p = params
    f32 = jnp.float32
    bf16 = jnp.bfloat16
    BN = N // _NB
    nb, nc = _NB, _NCACHE

    tgl = jnp.tanh(p["gate_local"]).reshape(1, 1)
    tgr = jnp.tanh(p["gate_return"]).reshape(1, 1)

    def h_idx(i):
        return (jnp.where(i < nb, i, jnp.where(i < nb + nc, nb - 1, i - nb)),
                0)

    const = lambda shape: pl.BlockSpec(shape, lambda i: (0,) * len(shape))

    x_out, x1out = pl.pallas_call(
        _mega_body,
        grid=(2 * _NB,),
        in_specs=[
            pl.BlockSpec((BN, M), h_idx),
            pl.BlockSpec((BN, D), lambda i: (jax.lax.rem(i, nb), 0)),
            const((M, D)),
            const((D, D)),
            const((1, D)),
            const((D, D)),
            const((1, D)),
            const((1, 1)),
            const((1, 1)),
            const((1, D)),
            const((1, D)),
            const((1, D)),
            const((1, D)),
            const((D, 2 * D)),
            const((1, 2 * D)),
            const((2 * D, D)),
            const((1, D)),
            const((D, D)),
            const((1, D)),
        ],
        out_specs=[
            pl.BlockSpec(
                (BN, D),
                lambda i: (jnp.where(i < nb, 0, i - nb), 0)),
            const((M, D)),
        ],
        out_shape=[
            jax.ShapeDtypeStruct((N, D), f32),
            jax.ShapeDtypeStruct((M, D), f32),
        ],
        scratch_shapes=[
            pltpu.VMEM((_NCACHE, BN, M), bf16),
            pltpu.VMEM((D, M), f32),
            pltpu.VMEM((1, M), f32),
            pltpu.VMEM((M, D), f32),
            pltpu.VMEM((M, D), bf16),
            pltpu.VMEM((1, M), f32),
            pltpu.VMEM((D, M), f32),
        ],
        compiler_params=pltpu.CompilerParams(
            dimension_semantics=("arbitrary",),
            vmem_limit_bytes=67108864,
        ),
    )(incidence_1, x_0, x_1,
      p["W_he"], p["b_he"].reshape(1, D), p["W_v"], p["b_v"].reshape(1, D),
      tgl, tgr,
      p["n1_g"].reshape(1, D), p["n1_b"].reshape(1, D),
      p["n2_g"].reshape(1, D), p["n2_b"].reshape(1, D),
      p["W1"].astype(bf16), p["b1"].reshape(1, 2 * D),
      p["W2"].astype(bf16), p["b2"].reshape(1, D),
      p["W_ret"], p["b_ret"].reshape(1, D))

    return x_out, x1out
